# baseline (speedup 1.0000x reference)
import functools

import jax
import jax.numpy as jnp
from jax.experimental import pallas as pl
from jax.experimental.pallas import tpu as pltpu

N_NODES = 100000
D_FEAT = 128
D_HID = 512
D_ATT = 256
N_CLASSES = 4
N_SEG = 16

TILE = 5000
NT = N_NODES // TILE


def _fused_kernel(x_ref, brow_ref, whT_ref, bh_ref, wabT_ref, bab_ref,
                  wc_ref, bcs_ref, wclsT_ref, bcls_ref,
                  out_ref, den_acc, pooled_acc):
    i = pl.program_id(0)

    @pl.when(i == 0)
    def _init():
        den_acc[...] = jnp.zeros((N_SEG, 1), dtype=jnp.float32)
        pooled_acc[...] = jnp.zeros((N_SEG, D_HID), dtype=jnp.float32)

    x_t = x_ref[...]
    h = jnp.maximum(
        jax.lax.dot_general(x_t, whT_ref[...], (((1,), (0,)), ((), ())),
                            preferred_element_type=jnp.float32)
        + bh_ref[...], 0.0)
    ab = (jax.lax.dot_general(h, wabT_ref[...], (((1,), (0,)), ((), ())),
                              preferred_element_type=jnp.float32)
          + bab_ref[...])
    ag = jnp.tanh(ab[:, :D_ATT]) * jax.nn.sigmoid(ab[:, D_ATT:])
    gate = (jax.lax.dot_general(wc_ref[...], ag, (((1,), (1,)), ((), ())),
                                preferred_element_type=jnp.float32)
            + bcs_ref[...])
    e_row = jnp.exp(gate)

    brow = brow_ref[0]
    seg = jax.lax.broadcasted_iota(jnp.int32, (N_SEG, TILE), 0).astype(
        jnp.float32)
    e_w = jnp.where(brow == seg, e_row, 0.0)

    den_tile = jnp.sum(e_w, axis=1, keepdims=True)
    contrib = jax.lax.dot_general(e_w, h, (((1,), (0,)), ((), ())),
                                  preferred_element_type=jnp.float32)

    den_acc[...] = den_acc[...] + den_tile
    pooled_acc[...] = pooled_acc[...] + contrib

    @pl.when(i == NT - 1)
    def _finish():
        den = den_acc[...]
        recip = jnp.where(den > 0, 1.0 / den, 0.0)
        pooled = pooled_acc[...] * recip
        out_ref[...] = (
            jax.lax.dot_general(pooled, wclsT_ref[...],
                                (((1,), (0,)), ((), ())),
                                preferred_element_type=jnp.float32)
            + bcls_ref[...])


@functools.partial(jax.jit, static_argnames=())
def kernel(x, edge_index, batch, W_head, b_head, Wa, ba, Wb, bb, Wc, bc,
           W_cls, b_cls):
    del edge_index
    brow = batch.astype(jnp.float32).reshape(NT, 1, TILE)
    whT = W_head.T
    wabT = jnp.concatenate([Wa.T, Wb.T], axis=1)
    bab = jnp.concatenate([ba, bb])[None, :]
    wclsT = W_cls.T
    bh = b_head[None, :]
    shift = jnp.sum(jnp.abs(Wc)) + jnp.abs(bc[0])
    bcs = (bc - shift)[None, :]
    bcls2 = b_cls[None, :]

    out = pl.pallas_call(
        _fused_kernel,
        grid=(NT,),
        in_specs=[
            pl.BlockSpec((TILE, D_FEAT), lambda i: (i, 0)),
            pl.BlockSpec((1, 1, TILE), lambda i: (i, 0, 0)),
            pl.BlockSpec((D_FEAT, D_HID), lambda i: (0, 0)),
            pl.BlockSpec((1, D_HID), lambda i: (0, 0)),
            pl.BlockSpec((D_HID, 2 * D_ATT), lambda i: (0, 0)),
            pl.BlockSpec((1, 2 * D_ATT), lambda i: (0, 0)),
            pl.BlockSpec((1, D_ATT), lambda i: (0, 0)),
            pl.BlockSpec((1, 1), lambda i: (0, 0)),
            pl.BlockSpec((D_HID, N_CLASSES), lambda i: (0, 0)),
            pl.BlockSpec((1, N_CLASSES), lambda i: (0, 0)),
        ],
        out_specs=pl.BlockSpec((N_SEG, N_CLASSES), lambda i: (0, 0)),
        out_shape=jax.ShapeDtypeStruct((N_SEG, N_CLASSES), jnp.float32),
        scratch_shapes=[
            pltpu.VMEM((N_SEG, 1), jnp.float32),
            pltpu.VMEM((N_SEG, D_HID), jnp.float32),
        ],
    )(x, brow, whT, bh, wabT, bab, Wc, bcs, wclsT, bcls2)
    return out


# speedup vs baseline: 1.0058x; 1.0058x over previous
import functools

import jax
import jax.numpy as jnp
from jax.experimental import pallas as pl
from jax.experimental.pallas import tpu as pltpu

N_NODES = 100000
D_FEAT = 128
D_HID = 512
D_ATT = 256
N_CLASSES = 4
N_SEG = 16

TILE = 4000
NT = N_NODES // TILE


def _fused_kernel(x_ref, brow_ref, whT_ref, bh_ref, wabT_ref, bab_ref,
                  wc_ref, bcs_ref, wclsT_ref, bcls_ref,
                  out_ref, den_acc, pooled_acc):
    i = pl.program_id(0)

    @pl.when(i == 0)
    def _init():
        den_acc[...] = jnp.zeros((N_SEG, 1), dtype=jnp.float32)
        pooled_acc[...] = jnp.zeros((N_SEG, D_HID), dtype=jnp.float32)

    x_t = x_ref[...]
    h = jnp.maximum(
        jax.lax.dot_general(x_t, whT_ref[...], (((1,), (0,)), ((), ())),
                            preferred_element_type=jnp.float32)
        + bh_ref[...], 0.0)
    ab = (jax.lax.dot_general(h, wabT_ref[...], (((1,), (0,)), ((), ())),
                              preferred_element_type=jnp.float32)
          + bab_ref[...])
    sig = 0.5 + 0.5 * jnp.tanh(0.5 * ab[:, D_ATT:])
    ag = jnp.tanh(ab[:, :D_ATT]) * sig
    gate = (jax.lax.dot_general(wc_ref[...], ag, (((1,), (1,)), ((), ())),
                                preferred_element_type=jnp.float32)
            + bcs_ref[...])
    e_row = jnp.exp(gate)

    brow = brow_ref[0]
    seg = jax.lax.broadcasted_iota(jnp.int32, (N_SEG, TILE), 0).astype(
        jnp.float32)
    e_w = jnp.where(brow == seg, e_row, 0.0)

    den_tile = jnp.sum(e_w, axis=1, keepdims=True)
    contrib = jax.lax.dot_general(e_w, h, (((1,), (0,)), ((), ())),
                                  preferred_element_type=jnp.float32)

    den_acc[...] = den_acc[...] + den_tile
    pooled_acc[...] = pooled_acc[...] + contrib

    @pl.when(i == NT - 1)
    def _finish():
        den = den_acc[...]
        recip = jnp.where(den > 0, 1.0 / den, 0.0)
        pooled = pooled_acc[...] * recip
        out_ref[...] = (
            jax.lax.dot_general(pooled, wclsT_ref[...],
                                (((1,), (0,)), ((), ())),
                                preferred_element_type=jnp.float32)
            + bcls_ref[...])


@functools.partial(jax.jit, static_argnames=())
def kernel(x, edge_index, batch, W_head, b_head, Wa, ba, Wb, bb, Wc, bc,
           W_cls, b_cls):
    del edge_index
    brow = batch.astype(jnp.float32).reshape(NT, 1, TILE)
    whT = W_head.T
    wabT = jnp.concatenate([Wa.T, Wb.T], axis=1)
    bab = jnp.concatenate([ba, bb])[None, :]
    wclsT = W_cls.T
    bh = b_head[None, :]
    shift = jnp.sum(jnp.abs(Wc)) + jnp.abs(bc[0])
    bcs = (bc - shift)[None, :]
    bcls2 = b_cls[None, :]

    out = pl.pallas_call(
        _fused_kernel,
        grid=(NT,),
        in_specs=[
            pl.BlockSpec((TILE, D_FEAT), lambda i: (i, 0)),
            pl.BlockSpec((1, 1, TILE), lambda i: (i, 0, 0)),
            pl.BlockSpec((D_FEAT, D_HID), lambda i: (0, 0)),
            pl.BlockSpec((1, D_HID), lambda i: (0, 0)),
            pl.BlockSpec((D_HID, 2 * D_ATT), lambda i: (0, 0)),
            pl.BlockSpec((1, 2 * D_ATT), lambda i: (0, 0)),
            pl.BlockSpec((1, D_ATT), lambda i: (0, 0)),
            pl.BlockSpec((1, 1), lambda i: (0, 0)),
            pl.BlockSpec((D_HID, N_CLASSES), lambda i: (0, 0)),
            pl.BlockSpec((1, N_CLASSES), lambda i: (0, 0)),
        ],
        out_specs=pl.BlockSpec((N_SEG, N_CLASSES), lambda i: (0, 0)),
        out_shape=jax.ShapeDtypeStruct((N_SEG, N_CLASSES), jnp.float32),
        scratch_shapes=[
            pltpu.VMEM((N_SEG, 1), jnp.float32),
            pltpu.VMEM((N_SEG, D_HID), jnp.float32),
        ],
    )(x, brow, whT, bh, wabT, bab, Wc, bcs, wclsT, bcls2)
    return out


# speedup vs baseline: 1.0850x; 1.0787x over previous
import functools

import jax
import jax.numpy as jnp
from jax.experimental import pallas as pl
from jax.experimental.pallas import tpu as pltpu

N_NODES = 100000
D_FEAT = 128
D_HID = 512
D_ATT = 256
N_CLASSES = 4
N_SEG = 16

TILE = 4000
NT = N_NODES // TILE


def _fused_kernel(x_ref, brow_ref, wh_ref, bh_ref, wa_ref, wb_ref, bab_ref,
                  wc_ref, bcs_ref, wcls_ref, bcls_ref,
                  out_ref, den_acc, pooled_acc):
    i = pl.program_id(0)

    @pl.when(i == 0)
    def _init():
        den_acc[...] = jnp.zeros((N_SEG, 1), dtype=jnp.float32)
        pooled_acc[...] = jnp.zeros((N_SEG, D_HID), dtype=jnp.float32)

    x_t = x_ref[...]
    h = jnp.maximum(
        jax.lax.dot_general(x_t, wh_ref[...], (((1,), (1,)), ((), ())),
                            preferred_element_type=jnp.float32)
        + bh_ref[...], 0.0)
    a_pre = jax.lax.dot_general(h, wa_ref[...], (((1,), (1,)), ((), ())),
                                preferred_element_type=jnp.float32)
    b_pre = jax.lax.dot_general(h, wb_ref[...], (((1,), (1,)), ((), ())),
                                preferred_element_type=jnp.float32)
    ab = (jnp.concatenate([a_pre, b_pre], axis=1)
          + bab_ref[...])
    sig = 0.5 + 0.5 * jnp.tanh(0.5 * ab[:, D_ATT:])
    ag = jnp.tanh(ab[:, :D_ATT]) * sig
    gate = (jax.lax.dot_general(wc_ref[...], ag, (((1,), (1,)), ((), ())),
                                preferred_element_type=jnp.float32)
            + bcs_ref[...])
    e_row = jnp.exp(gate)

    brow = brow_ref[0]
    seg = jax.lax.broadcasted_iota(jnp.int32, (N_SEG, TILE), 0)
    e_w = jnp.where(brow == seg, e_row, 0.0)

    den_tile = jnp.sum(e_w, axis=1, keepdims=True)
    contrib = jax.lax.dot_general(e_w, h, (((1,), (0,)), ((), ())),
                                  preferred_element_type=jnp.float32)

    den_acc[...] = den_acc[...] + den_tile
    pooled_acc[...] = pooled_acc[...] + contrib

    @pl.when(i == NT - 1)
    def _finish():
        den = den_acc[...]
        recip = jnp.where(den > 0, 1.0 / den, 0.0)
        pooled = pooled_acc[...] * recip
        out_ref[...] = (
            jax.lax.dot_general(pooled, wcls_ref[...],
                                (((1,), (1,)), ((), ())),
                                preferred_element_type=jnp.float32)
            + bcls_ref[...])


@functools.partial(jax.jit, static_argnames=())
def kernel(x, edge_index, batch, W_head, b_head, Wa, ba, Wb, bb, Wc, bc,
           W_cls, b_cls):
    del edge_index
    brow = batch.reshape(NT, 1, TILE)
    bab = jnp.concatenate([ba, bb])[None, :]
    bh = b_head[None, :]
    shift = jnp.sum(jnp.abs(Wc)) + jnp.abs(bc[0])
    bcs = (bc - shift)[None, :]
    bcls2 = b_cls[None, :]

    out = pl.pallas_call(
        _fused_kernel,
        grid=(NT,),
        in_specs=[
            pl.BlockSpec((TILE, D_FEAT), lambda i: (i, 0)),
            pl.BlockSpec((1, 1, TILE), lambda i: (i, 0, 0)),
            pl.BlockSpec((D_HID, D_FEAT), lambda i: (0, 0)),
            pl.BlockSpec((1, D_HID), lambda i: (0, 0)),
            pl.BlockSpec((D_ATT, D_HID), lambda i: (0, 0)),
            pl.BlockSpec((D_ATT, D_HID), lambda i: (0, 0)),
            pl.BlockSpec((1, 2 * D_ATT), lambda i: (0, 0)),
            pl.BlockSpec((1, D_ATT), lambda i: (0, 0)),
            pl.BlockSpec((1, 1), lambda i: (0, 0)),
            pl.BlockSpec((N_CLASSES, D_HID), lambda i: (0, 0)),
            pl.BlockSpec((1, N_CLASSES), lambda i: (0, 0)),
        ],
        out_specs=pl.BlockSpec((N_SEG, N_CLASSES), lambda i: (0, 0)),
        out_shape=jax.ShapeDtypeStruct((N_SEG, N_CLASSES), jnp.float32),
        scratch_shapes=[
            pltpu.VMEM((N_SEG, 1), jnp.float32),
            pltpu.VMEM((N_SEG, D_HID), jnp.float32),
        ],
    )(x, brow, W_head, bh, Wa, Wb, bab, Wc, bcs, W_cls, bcls2)
    return out
